# baseline (speedup 1.0000x reference)
import functools

import jax
import jax.numpy as jnp
from jax import lax
from jax.experimental import pallas as pl
from jax.experimental.pallas import tpu as pltpu
from jax.experimental.pallas import tpu_sc as plsc

D = 1024
SEQ = 64
NUM_WORKERS = 32
P_PER_W = SEQ // NUM_WORKERS
LANES = 16


def _pe_kernel(abs_hbm, file_hbm, rank_hbm, diag_hbm, anti_hbm, out_hbm,
               file_v, rank_v, diag_v, anti_v, abs_v, out_v, sem):
    wid = lax.axis_index("s") * 2 + lax.axis_index("c")
    p0 = wid * P_PER_W
    r = p0 // 8
    f = p0 % 8
    d = r + f
    a = r - f + 6

    copies = [
        pltpu.make_async_copy(file_hbm.at[pl.ds(f * D, 2 * D)], file_v, sem),
        pltpu.make_async_copy(rank_hbm.at[pl.ds(r * D, D)], rank_v, sem),
        pltpu.make_async_copy(diag_hbm.at[pl.ds(d * D, 2 * D)], diag_v, sem),
        pltpu.make_async_copy(anti_hbm.at[pl.ds(a * D, 2 * D)], anti_v, sem),
        pltpu.make_async_copy(abs_hbm.at[pl.ds(p0 * D, 2 * D)], abs_v, sem),
    ]
    for c in copies:
        c.start()
    for c in copies:
        c.wait()

    for p in range(P_PER_W):
        for j in range(D // LANES):
            sl = pl.ds(p * D + j * LANES, LANES)
            sl_r = pl.ds(j * LANES, LANES)
            sl_a = pl.ds((1 - p) * D + j * LANES, LANES)
            acc = file_v[sl] + rank_v[sl_r]
            acc = acc + diag_v[sl]
            acc = acc + anti_v[sl_a]
            acc = acc + abs_v[sl]
            out_v[sl] = acc

    pltpu.sync_copy(out_v, out_hbm.at[pl.ds(p0 * D, P_PER_W * D)])


@jax.jit
def _pe(abs_flat, file_flat, rank_flat, diag_flat, anti_flat):
    mesh = plsc.VectorSubcoreMesh(core_axis_name="c", subcore_axis_name="s")
    run = functools.partial(
        pl.kernel,
        mesh=mesh,
        out_type=jax.ShapeDtypeStruct((SEQ * D,), jnp.float32),
        scratch_types=[
            pltpu.VMEM((2 * D,), jnp.float32),
            pltpu.VMEM((D,), jnp.float32),
            pltpu.VMEM((2 * D,), jnp.float32),
            pltpu.VMEM((2 * D,), jnp.float32),
            pltpu.VMEM((2 * D,), jnp.float32),
            pltpu.VMEM((P_PER_W * D,), jnp.float32),
            pltpu.SemaphoreType.DMA,
        ],
    )(_pe_kernel)
    return run(abs_flat, file_flat, rank_flat, diag_flat, anti_flat)


def kernel(x, abs_pos, file_w, rank_w, diag_w, anti_w):
    del x
    out = _pe(abs_pos.reshape(-1), file_w.reshape(-1), rank_w.reshape(-1),
              diag_w.reshape(-1), anti_w.reshape(-1))
    return out.reshape(1, SEQ, D)
